# baseline (device time: 48221 ns/iter reference)
import jax
import jax.numpy as jnp
from jax import lax
from jax.experimental import pallas as pl
from jax.experimental.pallas import tpu as pltpu

N_DEV = 4
B, Sq, Hq, Dh = 2, 512, 8, 64
SKV = 512
D_MODEL = 768
D_QK = Hq * Dh
D_ST = 640
SCALE = 0.125

GLO = 32
TAIL = 128
STRIP = GLO + TAIL
MID0, MID1 = GLO, Sq - TAIL
MID = MID1 - MID0
P = MID // 2


def kernel(x, Wq, K_ext, V_ext, Wo):
    xb = x.astype(jnp.bfloat16)
    wqb = Wq.astype(jnp.bfloat16)
    wob = Wo.astype(jnp.bfloat16)
    kb = K_ext.reshape(B, SKV, D_QK).astype(jnp.bfloat16)
    vb = V_ext.reshape(B, SKV, D_QK).astype(jnp.bfloat16)

    def body(x_ref, wq_ref, k_ref, v_ref, wo_ref, out_ref,
             q_ref, sO_ref, sL_ref, sR_ref, sD_ref, mA_ref, mB_ref,
             ctx_ref, send_sems, recv_sems):
        my = lax.axis_index("i")
        left = lax.rem(my + N_DEV - 1, N_DEV)
        right = lax.rem(my + 1, N_DEV)

        for b in range(B):
            q_ref[b] = lax.dot_general(
                x_ref[b], wq_ref[...],
                (((1,), (0,)), ((), ())),
                preferred_element_type=jnp.float32,
            ).astype(jnp.bfloat16)

        off = my * SKV
        qi = lax.broadcasted_iota(jnp.int32, (Sq, SKV), 0)
        kg = lax.broadcasted_iota(jnp.int32, (Sq, SKV), 1) + off
        mask = (jnp.abs(qi - kg) <= 128) | (kg < 32) | (qi < 32)

        def state_rows(dst, dst0, qr0, qr1, msk):
            n = qr1 - qr0
            for b in range(B):
                for hd in range(Hq):
                    sl = slice(hd * Dh, (hd + 1) * Dh)
                    s = lax.dot_general(
                        q_ref[b, qr0:qr1, sl], k_ref[b, :, sl],
                        (((1,), (1,)), ((), ())),
                        preferred_element_type=jnp.float32,
                    ) * SCALE
                    p = jnp.where(msk, jnp.exp(s), 0.0)
                    dst[b, dst0:dst0 + n, D_QK + hd:D_QK + hd + 1] = jnp.sum(
                        p, axis=1, keepdims=True).astype(jnp.bfloat16)
                    dst[b, dst0:dst0 + n, sl] = lax.dot_general(
                        p.astype(jnp.bfloat16), v_ref[b, :, sl],
                        (((1,), (0,)), ((), ())),
                        preferred_element_type=jnp.float32,
                    ).astype(jnp.bfloat16)

        state_rows(sO_ref, 0, 0, GLO, mask[0:GLO])
        state_rows(sO_ref, GLO, MID1, Sq, mask[MID1:Sq])

        barrier = pltpu.get_barrier_semaphore()
        for nbr in (left, right):
            pl.semaphore_signal(barrier, inc=1, device_id=(nbr,),
                                device_id_type=pl.DeviceIdType.MESH)
        pl.semaphore_wait(barrier, 2)

        def rdma(i, src, dst, dev):
            return pltpu.make_async_remote_copy(
                src_ref=src, dst_ref=dst,
                send_sem=send_sems.at[i], recv_sem=recv_sems.at[i],
                device_id=(dev,), device_id_type=pl.DeviceIdType.MESH,
            )

        h1 = [
            rdma(0, sO_ref, sL_ref, right),
            rdma(1, sO_ref, sR_ref, left),
        ]
        for r in h1:
            r.start()

        @pl.when(my == 0)
        def _():
            state_rows(mA_ref, 0, MID0, MID0 + P, mask[MID0:MID0 + P])
            p1 = [
                rdma(4, mA_ref, mA_ref, right),
                rdma(8, mA_ref, mA_ref, left),
            ]
            for r in p1:
                r.start()
            state_rows(mB_ref, 0, MID0 + P, MID1, mask[MID0 + P:MID1])
            p2 = [
                rdma(6, mB_ref, mB_ref, right),
                rdma(9, mB_ref, mB_ref, left),
            ]
            for r in p2:
                r.start()
            for r in p1 + p2:
                r.wait_send()

        for r in h1:
            r.wait()

        h2 = [
            rdma(2, sL_ref.at[0:1], sD_ref.at[0:1], right),
            rdma(3, sR_ref.at[1:2], sD_ref.at[1:2], left),
        ]
        for r in h2:
            r.start()

        @pl.when(my == 1)
        def _():
            rdma(4, mA_ref, mA_ref, left).wait_recv()
            fwd = rdma(5, mA_ref, mA_ref, right)
            fwd.start()
            rdma(6, mB_ref, mB_ref, left).wait_recv()
            fwd.wait_send()

        @pl.when(my == 3)
        def _():
            rdma(9, mB_ref, mB_ref, right).wait_recv()
            fwd = rdma(7, mB_ref, mB_ref, left)
            fwd.start()
            rdma(8, mA_ref, mA_ref, right).wait_recv()
            fwd.wait_send()

        @pl.when(my == 2)
        def _():
            rdma(5, mA_ref, mA_ref, left).wait_recv()
            rdma(7, mB_ref, mB_ref, right).wait_recv()

        for r in h2:
            r.wait()

        for b in range(B):
            a = (sO_ref[b].astype(jnp.float32)
                 + sL_ref[b].astype(jnp.float32)
                 + sR_ref[b].astype(jnp.float32)
                 + sD_ref[b].astype(jnp.float32))
            midA = mA_ref[b].astype(jnp.float32)
            midB = mB_ref[b].astype(jnp.float32)
            for hd in range(Hq):
                sl = slice(hd * Dh, (hd + 1) * Dh)
                lc = slice(D_QK + hd, D_QK + hd + 1)
                ctx_ref[b, 0:GLO, sl] = (
                    a[0:GLO, sl] / a[0:GLO, lc]).astype(jnp.bfloat16)
                ctx_ref[b, MID1:Sq, sl] = (
                    a[GLO:STRIP, sl] / a[GLO:STRIP, lc]
                ).astype(jnp.bfloat16)
                ctx_ref[b, MID0:MID0 + P, sl] = (
                    midA[:, sl] / midA[:, lc]).astype(jnp.bfloat16)
                ctx_ref[b, MID0 + P:MID1, sl] = (
                    midB[:, sl] / midB[:, lc]).astype(jnp.bfloat16)
            out_ref[b] = lax.dot_general(
                ctx_ref[b], wo_ref[...], (((1,), (0,)), ((), ())),
                preferred_element_type=jnp.float32,
            ).astype(jnp.bfloat16)

    strip_shape = pltpu.VMEM((B, STRIP, D_ST), jnp.bfloat16)
    return pl.pallas_call(
        body,
        out_shape=jax.ShapeDtypeStruct((B, Sq, D_MODEL), jnp.bfloat16),
        in_specs=[pl.BlockSpec(memory_space=pltpu.VMEM)] * 5,
        out_specs=pl.BlockSpec(memory_space=pltpu.VMEM),
        scratch_shapes=[
            pltpu.VMEM((B, Sq, D_QK), jnp.bfloat16),
            strip_shape,
            strip_shape,
            strip_shape,
            strip_shape,
            pltpu.VMEM((B, P, D_ST), jnp.bfloat16),
            pltpu.VMEM((B, P, D_ST), jnp.bfloat16),
            pltpu.VMEM((B, Sq, D_QK), jnp.bfloat16),
            pltpu.SemaphoreType.DMA((10,)),
            pltpu.SemaphoreType.DMA((10,)),
        ],
        compiler_params=pltpu.CompilerParams(
            collective_id=0, vmem_limit_bytes=100 * 1024 * 1024),
    )(xb, wqb, kb, vb, wob)


# device time: 44689 ns/iter; 1.0790x vs baseline; 1.0790x over previous
import jax
import jax.numpy as jnp
from jax import lax
from jax.experimental import pallas as pl
from jax.experimental.pallas import tpu as pltpu

N_DEV = 4
B, Sq, Hq, Dh = 2, 512, 8, 64
SKV = 512
D_MODEL = 768
D_QK = Hq * Dh
D_ST = 640
SCALE = 0.125

GLO = 32
TAIL = 128
STRIP = GLO + TAIL
MID0, MID1 = GLO, Sq - TAIL
MID = MID1 - MID0
P = MID // 2


def kernel(x, Wq, K_ext, V_ext, Wo):
    kb = K_ext.reshape(B, SKV, D_QK)
    vb = V_ext.reshape(B, SKV, D_QK)

    def body(x_ref, wq_ref, k_ref, v_ref, wo_ref, out_ref,
             q_ref, sO_ref, sL_ref, sR_ref, sD_ref, mA_ref, mB_ref,
             ctx_ref, send_sems, recv_sems):
        my = lax.axis_index("i")
        left = lax.rem(my + N_DEV - 1, N_DEV)
        right = lax.rem(my + 1, N_DEV)

        wqc = wq_ref[...].astype(jnp.bfloat16)
        for b in range(B):
            q_ref[b] = lax.dot_general(
                x_ref[b].astype(jnp.bfloat16), wqc,
                (((1,), (0,)), ((), ())),
                preferred_element_type=jnp.float32,
            ).astype(jnp.bfloat16)

        off = my * SKV
        qi = lax.broadcasted_iota(jnp.int32, (Sq, SKV), 0)
        kg = lax.broadcasted_iota(jnp.int32, (Sq, SKV), 1) + off
        mask = (jnp.abs(qi - kg) <= 128) | (kg < 32) | (qi < 32)

        def state_rows(dst, dst0, qr0, qr1, msk):
            n = qr1 - qr0
            for b in range(B):
                for hd in range(Hq):
                    sl = slice(hd * Dh, (hd + 1) * Dh)
                    s = lax.dot_general(
                        q_ref[b, qr0:qr1, sl],
                        k_ref[b, :, sl].astype(jnp.bfloat16),
                        (((1,), (1,)), ((), ())),
                        preferred_element_type=jnp.float32,
                    ) * SCALE
                    p = jnp.where(msk, jnp.exp(s), 0.0)
                    dst[b, dst0:dst0 + n, D_QK + hd:D_QK + hd + 1] = jnp.sum(
                        p, axis=1, keepdims=True).astype(jnp.bfloat16)
                    dst[b, dst0:dst0 + n, sl] = lax.dot_general(
                        p.astype(jnp.bfloat16),
                        v_ref[b, :, sl].astype(jnp.bfloat16),
                        (((1,), (0,)), ((), ())),
                        preferred_element_type=jnp.float32,
                    ).astype(jnp.bfloat16)

        state_rows(sO_ref, 0, 0, GLO, mask[0:GLO])
        state_rows(sO_ref, GLO, MID1, Sq, mask[MID1:Sq])

        barrier = pltpu.get_barrier_semaphore()
        for nbr in (left, right):
            pl.semaphore_signal(barrier, inc=1, device_id=(nbr,),
                                device_id_type=pl.DeviceIdType.MESH)
        pl.semaphore_wait(barrier, 2)

        def rdma(i, src, dst, dev):
            return pltpu.make_async_remote_copy(
                src_ref=src, dst_ref=dst,
                send_sem=send_sems.at[i], recv_sem=recv_sems.at[i],
                device_id=(dev,), device_id_type=pl.DeviceIdType.MESH,
            )

        h1 = [
            rdma(0, sO_ref, sL_ref, right),
            rdma(1, sO_ref, sR_ref, left),
        ]
        for r in h1:
            r.start()

        @pl.when(my == 0)
        def _():
            state_rows(mA_ref, 0, MID0, MID0 + P, mask[MID0:MID0 + P])
            p1 = [
                rdma(4, mA_ref, mA_ref, right),
                rdma(8, mA_ref, mA_ref, left),
            ]
            for r in p1:
                r.start()
            state_rows(mB_ref, 0, MID0 + P, MID1, mask[MID0 + P:MID1])
            p2 = [
                rdma(6, mB_ref, mB_ref, right),
                rdma(9, mB_ref, mB_ref, left),
            ]
            for r in p2:
                r.start()
            for r in p1 + p2:
                r.wait_send()

        for r in h1:
            r.wait()

        h2 = [
            rdma(2, sL_ref.at[0:1], sD_ref.at[0:1], right),
            rdma(3, sR_ref.at[1:2], sD_ref.at[1:2], left),
        ]
        for r in h2:
            r.start()

        @pl.when(my == 1)
        def _():
            rdma(4, mA_ref, mA_ref, left).wait_recv()
            fwd = rdma(5, mA_ref, mA_ref, right)
            fwd.start()
            rdma(6, mB_ref, mB_ref, left).wait_recv()
            fwd.wait_send()

        @pl.when(my == 3)
        def _():
            rdma(9, mB_ref, mB_ref, right).wait_recv()
            fwd = rdma(7, mB_ref, mB_ref, left)
            fwd.start()
            rdma(8, mA_ref, mA_ref, right).wait_recv()
            fwd.wait_send()

        @pl.when(my == 2)
        def _():
            rdma(5, mA_ref, mA_ref, left).wait_recv()
            rdma(7, mB_ref, mB_ref, right).wait_recv()

        for r in h2:
            r.wait()

        for b in range(B):
            a = (sO_ref[b].astype(jnp.float32)
                 + sL_ref[b].astype(jnp.float32)
                 + sR_ref[b].astype(jnp.float32)
                 + sD_ref[b].astype(jnp.float32))
            midA = mA_ref[b].astype(jnp.float32)
            midB = mB_ref[b].astype(jnp.float32)
            for hd in range(Hq):
                sl = slice(hd * Dh, (hd + 1) * Dh)
                lc = slice(D_QK + hd, D_QK + hd + 1)
                ctx_ref[b, 0:GLO, sl] = (
                    a[0:GLO, sl] / a[0:GLO, lc]).astype(jnp.bfloat16)
                ctx_ref[b, MID1:Sq, sl] = (
                    a[GLO:STRIP, sl] / a[GLO:STRIP, lc]
                ).astype(jnp.bfloat16)
                ctx_ref[b, MID0:MID0 + P, sl] = (
                    midA[:, sl] / midA[:, lc]).astype(jnp.bfloat16)
                ctx_ref[b, MID0 + P:MID1, sl] = (
                    midB[:, sl] / midB[:, lc]).astype(jnp.bfloat16)
            out_ref[b] = lax.dot_general(
                ctx_ref[b], wo_ref[...].astype(jnp.bfloat16),
                (((1,), (0,)), ((), ())),
                preferred_element_type=jnp.float32,
            ).astype(jnp.bfloat16)

    strip_shape = pltpu.VMEM((B, STRIP, D_ST), jnp.bfloat16)
    return pl.pallas_call(
        body,
        out_shape=jax.ShapeDtypeStruct((B, Sq, D_MODEL), jnp.bfloat16),
        in_specs=[pl.BlockSpec(memory_space=pltpu.VMEM)] * 5,
        out_specs=pl.BlockSpec(memory_space=pltpu.VMEM),
        scratch_shapes=[
            pltpu.VMEM((B, Sq, D_QK), jnp.bfloat16),
            strip_shape,
            strip_shape,
            strip_shape,
            strip_shape,
            pltpu.VMEM((B, P, D_ST), jnp.bfloat16),
            pltpu.VMEM((B, P, D_ST), jnp.bfloat16),
            pltpu.VMEM((B, Sq, D_QK), jnp.bfloat16),
            pltpu.SemaphoreType.DMA((10,)),
            pltpu.SemaphoreType.DMA((10,)),
        ],
        compiler_params=pltpu.CompilerParams(
            collective_id=0, vmem_limit_bytes=100 * 1024 * 1024),
    )(x, Wq, kb, vb, Wo)
